# baseline (speedup 1.0000x reference)
import functools

import jax
import jax.numpy as jnp
from jax import lax
from jax.experimental import pallas as pl
from jax.experimental.pallas import tpu as pltpu
from jax.experimental.pallas import tpu_sc as plsc

NUM_CHANNELS = 100000
EMBED_DIM = 64
BATCH = 16384
DATE_DIM = 5

NC = 2
NS = 16
L = 16
NW = NC * NS
DPW = EMBED_DIM // NW
KC = 2048
NKC = BATCH // KC

CB = 4096

_mesh = plsc.VectorSubcoreMesh(core_axis_name="c", subcore_axis_name="s")


def _date_body(w_ref, dt_ref, b_ref, out_ref):
    de = lax.dot_general(w_ref[...], dt_ref[...], (((1,), (0,)), ((), ())),
                         preferred_element_type=jnp.float32)
    out_ref[0:EMBED_DIM, :] = de + b_ref[...]


_date_proj = pl.pallas_call(
    _date_body,
    out_shape=jax.ShapeDtypeStruct((2 * EMBED_DIM, BATCH), jnp.float32),
    grid=(BATCH // CB,),
    in_specs=[
        pl.BlockSpec((EMBED_DIM, DATE_DIM), lambda i: (0, 0)),
        pl.BlockSpec((DATE_DIM, CB), lambda i: (0, i)),
        pl.BlockSpec((EMBED_DIM, 1), lambda i: (0, 0)),
    ],
    out_specs=pl.BlockSpec((2 * EMBED_DIM, CB), lambda i: (0, i)),
)


@functools.partial(
    pl.kernel,
    mesh=_mesh,
    out_type=(),
    scratch_types=[
        pltpu.VMEM((NUM_CHANNELS,), jnp.float32),
        pltpu.VMEM((KC,), jnp.int32),
        pltpu.VMEM((KC,), jnp.float32),
    ],
    compiler_params=pltpu.CompilerParams(use_tc_tiling_on_sc=False,
                                         needs_layout_passes=False),
)
def _sc_gather(idx_hbm, tflat_hbm, out_hbm, row_v, idxc_v, outc_v):
    wid = lax.axis_index("s") * NC + lax.axis_index("c")

    for r in range(DPW):
        d = wid * DPW + r
        pltpu.sync_copy(tflat_hbm.at[pl.ds(d * NUM_CHANNELS, NUM_CHANNELS)],
                        row_v)

        def chunk_body(c, carry):
            pltpu.sync_copy(idx_hbm.at[pl.ds(c * KC, KC)], idxc_v)

            def g_body(g, carry2):
                iv = idxc_v[pl.ds(g * L, L)]
                outc_v[pl.ds(g * L, L)] = plsc.load_gather(row_v, [iv])
                return carry2

            lax.fori_loop(0, KC // L, g_body, 0)
            pltpu.sync_copy(outc_v,
                            out_hbm.at[EMBED_DIM + d, pl.ds(c * KC, KC)])
            return carry

        lax.fori_loop(0, NKC, chunk_body, 0)


def kernel(date_features, channel_ids, channel_table, date_W, date_b):
    outT0 = _date_proj(date_W, date_features.T, date_b.reshape(EMBED_DIM, 1))
    out_ref = jax.new_ref(outT0)
    tflat = channel_table.T.reshape(EMBED_DIM * NUM_CHANNELS)
    _sc_gather(channel_ids.astype(jnp.int32), tflat, out_ref)
    return out_ref[...].T


# speedup vs baseline: 1.6070x; 1.6070x over previous
import functools

import jax
import jax.numpy as jnp
from jax import lax
from jax.experimental import pallas as pl
from jax.experimental.pallas import tpu as pltpu
from jax.experimental.pallas import tpu_sc as plsc

NUM_CHANNELS = 100000
EMBED_DIM = 64
BATCH = 16384
DATE_DIM = 5

NC = 2
NS = 16
L = 16
NW = NC * NS
BPW = BATCH // NW
HALF = BPW // 2
CHUNK = 128
DVEC = EMBED_DIM // L

RBT = 2048
NBLK = 25
SPLIT = NBLK * RBT

_mesh = plsc.VectorSubcoreMesh(core_axis_name="c", subcore_axis_name="s")


def _pack_body(lo_ref, hi_ref, out_ref):
    out_ref[:, 0:EMBED_DIM] = lo_ref[...].T
    out_ref[:, EMBED_DIM:2 * EMBED_DIM] = hi_ref[...].T


_pack_table = pl.pallas_call(
    _pack_body,
    out_shape=jax.ShapeDtypeStruct((SPLIT, 2 * EMBED_DIM), jnp.float32),
    grid=(NBLK,),
    in_specs=[
        pl.BlockSpec((EMBED_DIM, RBT), lambda i: (0, i)),
        pl.BlockSpec((EMBED_DIM, RBT), lambda i: (0, jnp.minimum(i + NBLK,
                                                                 2 * NBLK - 2))),
    ],
    out_specs=pl.BlockSpec((RBT, 2 * EMBED_DIM), lambda i: (i, 0)),
)


@functools.partial(
    pl.kernel,
    mesh=_mesh,
    out_type=jax.ShapeDtypeStruct((BATCH, 2 * EMBED_DIM), jnp.float32),
    scratch_types=[
        pltpu.VMEM((BPW,), jnp.int32),
        pltpu.VMEM((HALF,), jnp.int32),
        pltpu.VMEM((HALF, 2 * EMBED_DIM), jnp.float32),
        pltpu.VMEM((HALF, 2 * EMBED_DIM), jnp.float32),
        pltpu.VMEM((DATE_DIM, BPW), jnp.float32),
        pltpu.VMEM((DATE_DIM, EMBED_DIM), jnp.float32),
        pltpu.VMEM((EMBED_DIM,), jnp.float32),
        pltpu.SemaphoreType.DMA,
    ],
    compiler_params=pltpu.CompilerParams(use_tc_tiling_on_sc=False),
)
def _sc_main(date_hbm, idx_hbm, pairs_hbm, w_hbm, bias_hbm, out_hbm,
             idx_v, pidx_v, rows_v, comb_v, date_v, w_v, bias_v, gsem):
    wid = lax.axis_index("s") * NC + lax.axis_index("c")
    base = wid * BPW

    pltpu.sync_copy(idx_hbm.at[pl.ds(base, BPW)], idx_v)
    pltpu.sync_copy(date_hbm.at[:, pl.ds(base, BPW)], date_v)
    pltpu.sync_copy(w_hbm, w_v)
    pltpu.sync_copy(bias_hbm, bias_v)

    wvec = [[w_v[k, pl.ds(d * L, L)] for d in range(DVEC)]
            for k in range(DATE_DIM)]
    bvec = [bias_v[pl.ds(d * L, L)] for d in range(DVEC)]

    for h in range(2):
        hb = h * HALF

        def shift_body(g, carry):
            pidx_v[pl.ds(g * L, L)] = lax.rem(
                idx_v[pl.ds(hb + g * L, L)], SPLIT)
            return carry

        lax.fori_loop(0, HALF // L, shift_body, 0)

        copies = []
        for j in range(HALF // CHUNK):
            copies.append(
                pltpu.async_copy(
                    pairs_hbm.at[pidx_v.at[pl.ds(j * CHUNK, CHUNK)]],
                    rows_v.at[pl.ds(j * CHUNK, CHUNK)],
                    gsem,
                )
            )

        def group_body(g, carry):
            sv = [date_v[k, pl.ds(hb + g * L, L)] for k in range(DATE_DIM)]
            for r in range(L):
                b = g * L + r
                for d in range(DVEC):
                    acc = bvec[d]
                    for k in range(DATE_DIM):
                        acc = acc + sv[k][r] * wvec[k][d]
                    comb_v[b, pl.ds(d * L, L)] = acc
            return carry

        lax.fori_loop(0, HALF // L, group_body, 0)

        for c in copies:
            c.wait()

        def sel_body(g, carry):
            iv = idx_v[pl.ds(hb + g * L, L)]
            hv = jnp.where(iv >= SPLIT, 1, 0)
            for r in range(L):
                b = g * L + r
                take_hi = hv[r] > 0
                for d in range(DVEC):
                    lo = rows_v[b, pl.ds(d * L, L)]
                    hi = rows_v[b, pl.ds(EMBED_DIM + d * L, L)]
                    comb_v[b, pl.ds(EMBED_DIM + d * L, L)] = jnp.where(
                        take_hi, hi, lo)
            return carry

        lax.fori_loop(0, HALF // L, sel_body, 0)

        pltpu.sync_copy(comb_v, out_hbm.at[pl.ds(base + hb, HALF)])


def kernel(date_features, channel_ids, channel_table, date_W, date_b):
    tt = channel_table.T
    pairs = _pack_table(tt, tt)
    return _sc_main(date_features.T, channel_ids.astype(jnp.int32), pairs,
                    date_W.T, date_b)
